# baseline (speedup 1.0000x reference)
import functools

import jax
import jax.numpy as jnp
from jax import lax
from jax.experimental import pallas as pl
from jax.experimental.pallas import tpu as pltpu
from jax.experimental.pallas import tpu_sc as plsc

N = 10000
F = 128
H = 8
C = 16
NPAD = 10240
NT = 16
RPT = NPAD // NT
CH = 128
NW = 32
PW = 84
NB = 4
EPC = PW * NW
EP = EPC * CH
BLK = 512

_SC_PARAMS = dict(
    compiler_params=pltpu.CompilerParams(use_tc_tiling_on_sc=False,
                                         needs_layout_passes=False),
)


def _mesh():
    return plsc.VectorSubcoreMesh(core_axis_name="c", subcore_axis_name="s")


def _sc_edge_logits(src2, dst2, asn, adn, z16):
    @functools.partial(
        pl.kernel,
        out_type=(jax.ShapeDtypeStruct((EP, C), jnp.float32),
                  jax.ShapeDtypeStruct((2, NPAD, C), jnp.float32)),
        mesh=_mesh(),
        scratch_types=[
            pltpu.VMEM((PW, CH), jnp.int32),
            pltpu.VMEM((PW, CH), jnp.int32),
            pltpu.VMEM((NB, CH, C), jnp.float32),
            pltpu.VMEM((NB, CH, C), jnp.float32),
            pltpu.VMEM_SHARED((NPAD, C), jnp.float32),
            pltpu.SemaphoreType.DMA((NB,)),
            pltpu.SemaphoreType.DMA((NB,)),
            pltpu.SemaphoreType.DMA((NB,)),
            pltpu.SemaphoreType.DMA((NB,)),
        ],
        **_SC_PARAMS,
    )
    def k(src_h, dst_h, as_h, ad_h, z_h, e_h, esum_h, isrc, idst, ba, bb,
          acc, sga, sgb, sse, ssc):
        cid = lax.axis_index("c")
        sid = lax.axis_index("s")
        wid = sid * 2 + cid
        pltpu.sync_copy(z_h, acc.at[pl.ds(sid * RPT, RPT)])
        pltpu.sync_copy(src_h.at[pl.ds(wid * PW, PW)], isrc)
        pltpu.sync_copy(dst_h.at[pl.ds(wid * PW, PW)], idst)
        plsc.subcore_barrier()

        def issue_g(j, k_):
            pltpu.async_copy(as_h.at[isrc.at[j]], ba.at[k_], sga.at[k_])
            pltpu.async_copy(ad_h.at[idst.at[j]], bb.at[k_], sgb.at[k_])

        def wait_g(j, k_):
            pltpu.make_async_copy(as_h.at[isrc.at[j]], ba.at[k_],
                                  sga.at[k_]).wait()
            pltpu.make_async_copy(ad_h.at[idst.at[j]], bb.at[k_],
                                  sgb.at[k_]).wait()

        def issue_s(j, k_):
            base = pl.multiple_of((wid * PW + j) * CH, CH)
            pltpu.async_copy(ba.at[k_], e_h.at[pl.ds(base, CH)],
                             sse.at[k_])
            pltpu.async_copy(ba.at[k_], acc.at[idst.at[j]], ssc.at[k_],
                             add=True)

        def wait_s(j, k_):
            base = pl.multiple_of((wid * PW + j) * CH, CH)
            pltpu.make_async_copy(ba.at[k_], e_h.at[pl.ds(base, CH)],
                                  sse.at[k_]).wait()
            pltpu.make_async_copy(ba.at[k_], acc.at[idst.at[j]],
                                  ssc.at[k_]).wait()

        def compute(k_):
            def row_body(r, _):
                v = ba[k_, r, :] + bb[k_, r, :]
                v = jnp.where(v >= 0.0, v, v * 0.2)
                ba[k_, r, :] = jnp.exp(v)
                return 0

            lax.fori_loop(0, CH, row_body, 0, unroll=4)

        def step(j, k_, drain_prev, prefetch):
            wait_g(j, k_)
            compute(k_)
            issue_s(j, k_)
            k2 = (k_ + 2) % NB
            if drain_prev:
                wait_s(j - 2, k2)
            if prefetch:
                issue_g(j + 2, k2)

        issue_g(0, 0)
        issue_g(1, 1)
        for k_ in range(NB):
            step(k_, k_, k_ >= 2, True)

        def outer(i, _):
            for k_ in range(NB):
                step(i * NB + k_, k_, True, True)
            return 0

        lax.fori_loop(1, PW // NB - 1, outer, 0)
        for k_ in range(NB):
            j = PW - NB + k_
            step(j, k_, True, j + 2 < PW)
        wait_s(PW - 2, (PW - 2) % NB)
        wait_s(PW - 1, (PW - 1) % NB)
        plsc.subcore_barrier()
        pltpu.sync_copy(acc.at[pl.ds(sid * RPT, RPT)],
                        esum_h.at[cid, pl.ds(sid * RPT, RPT)])

    return k(src2, dst2, asn, adn, z16)


CH2 = 64
PW2 = PW * 2
EPC2 = EPC * 2


def _sc_aggregate(src2, dst2, e_all, invn, hn, z128):
    @functools.partial(
        pl.kernel,
        out_type=(jax.ShapeDtypeStruct((EP, H), jnp.float32),
                  jax.ShapeDtypeStruct((2, NPAD, F), jnp.float32)),
        mesh=_mesh(),
        scratch_types=[
            pltpu.VMEM((4, CH2), jnp.int32),
            pltpu.VMEM((4, CH2), jnp.int32),
            pltpu.VMEM((3, CH2, C), jnp.float32),
            pltpu.VMEM((3, CH2, C), jnp.float32),
            pltpu.VMEM((3, CH2, F), jnp.float32),
            pltpu.VMEM_SHARED((NPAD, F), jnp.float32),
            pltpu.SemaphoreType.DMA((4,)),
            pltpu.SemaphoreType.DMA((4,)),
            pltpu.SemaphoreType.DMA((3,)),
            pltpu.SemaphoreType.DMA((3,)),
            pltpu.SemaphoreType.DMA((3,)),
            pltpu.SemaphoreType.DMA((3,)),
            pltpu.SemaphoreType.DMA((3,)),
        ],
        **_SC_PARAMS,
    )
    def k(src_h, dst_h, e_h, inv_h, h_h, z_h, alpha_h, accp_h, isrc, idst,
          be, binv, brows, acc, sis, sid_, sge, sgi, sgh, ssa, ssc):
        cid = lax.axis_index("c")
        sid = lax.axis_index("s")
        wid = sid * 2 + cid
        pltpu.sync_copy(z_h, acc.at[pl.ds(sid * RPT, RPT)])
        plsc.subcore_barrier()

        def issue_i(j, q):
            g = wid * PW2 + j
            pltpu.async_copy(src_h.at[g], isrc.at[q], sis.at[q])
            pltpu.async_copy(dst_h.at[g], idst.at[q], sid_.at[q])

        def wait_i(j, q):
            g = wid * PW2 + j
            pltpu.make_async_copy(src_h.at[g], isrc.at[q], sis.at[q]).wait()
            pltpu.make_async_copy(dst_h.at[g], idst.at[q],
                                  sid_.at[q]).wait()

        def issue_g(j, s, q):
            base = pl.multiple_of((wid * PW2 + j) * CH2, CH2)
            pltpu.async_copy(e_h.at[pl.ds(base, CH2)], be.at[s], sge.at[s])
            pltpu.async_copy(inv_h.at[idst.at[q]], binv.at[s], sgi.at[s])
            pltpu.async_copy(h_h.at[isrc.at[q]], brows.at[s], sgh.at[s])

        def wait_g(j, s, q):
            base = pl.multiple_of((wid * PW2 + j) * CH2, CH2)
            pltpu.make_async_copy(e_h.at[pl.ds(base, CH2)], be.at[s],
                                  sge.at[s]).wait()
            pltpu.make_async_copy(inv_h.at[idst.at[q]], binv.at[s],
                                  sgi.at[s]).wait()
            pltpu.make_async_copy(h_h.at[isrc.at[q]], brows.at[s],
                                  sgh.at[s]).wait()

        def issue_s(j, s, q):
            base = pl.multiple_of((wid * PW2 + j) * CH2, CH2)
            pltpu.async_copy(be.at[s, :, pl.ds(0, H)],
                             alpha_h.at[pl.ds(base, CH2)], ssa.at[s])
            pltpu.async_copy(brows.at[s], acc.at[idst.at[q]], ssc.at[s],
                             add=True)

        def wait_s(j, s, q):
            base = pl.multiple_of((wid * PW2 + j) * CH2, CH2)
            pltpu.make_async_copy(be.at[s, :, pl.ds(0, H)],
                                  alpha_h.at[pl.ds(base, CH2)],
                                  ssa.at[s]).wait()
            pltpu.make_async_copy(brows.at[s], acc.at[idst.at[q]],
                                  ssc.at[s]).wait()

        gdn = lax.GatherDimensionNumbers(offset_dims=(),
                                         collapsed_slice_dims=(0,),
                                         start_index_map=(0,))

        def compute(s):
            def row_body(r, _):
                va = be[s, r, :] * binv[s, r, :]
                be[s, r, :] = va
                for hh in range(H):
                    ah = lax.gather(
                        va, jnp.full((C, 1), hh, jnp.int32), gdn, (1,),
                        mode=lax.GatherScatterMode.PROMISE_IN_BOUNDS)
                    brows[s, r, pl.ds(hh * C, C)] = (
                        brows[s, r, pl.ds(hh * C, C)] * ah)
                return 0

            lax.fori_loop(0, CH2, row_body, 0, unroll=2)

        def step(j, k_, drain_prev, pf_g, pf_i):
            s = k_ % 3
            s1 = (k_ + 1) % 3
            q = k_ % 4
            q1 = (k_ + 1) % 4
            q2 = (k_ + 2) % 4
            wait_g(j, s, q)
            if pf_g:
                wait_i(j + 1, q1)
            if drain_prev:
                wait_s(j - 2, s1, q2)
            if pf_g:
                issue_g(j + 1, s1, q1)
            compute(s)
            issue_s(j, s, q)
            if pf_i:
                issue_i(j + 2, q2)

        issue_i(0, 0)
        issue_i(1, 1)
        wait_i(0, 0)
        issue_g(0, 0, 0)
        for k_ in range(12):
            step(k_, k_, k_ >= 2, True, True)

        def outer(i, _):
            for k_ in range(12):
                step(i * 12 + k_, k_, True, True, True)
            return 0

        lax.fori_loop(1, PW2 // 12 - 1, outer, 0)
        for k_ in range(12):
            j = PW2 - 12 + k_
            step(j, k_, True, j + 1 < PW2, j + 2 < PW2)
        wait_s(PW2 - 2, (PW2 - 2) % 3, (PW2 - 2) % 4)
        wait_s(PW2 - 1, (PW2 - 1) % 3, (PW2 - 1) % 4)
        plsc.subcore_barrier()
        pltpu.sync_copy(acc.at[pl.ds(sid * RPT, RPT)],
                        accp_h.at[cid, pl.ds(sid * RPT, RPT)])

    return k(src2, dst2, e_all, invn, hn, z128)


def _tc_project_body(x_ref, w_ref, as_ref, ad_ref, h_ref, asn_ref, adn_ref):
    h = jnp.dot(x_ref[...], w_ref[...], preferred_element_type=jnp.float32)
    h_ref[...] = h
    asn_ref[...] = jnp.dot(h, as_ref[...], preferred_element_type=jnp.float32)
    adn_ref[...] = jnp.dot(h, ad_ref[...], preferred_element_type=jnp.float32)


def _tc_project(xp, W, As, Ad):
    grid = (NPAD // BLK,)
    return pl.pallas_call(
        _tc_project_body,
        grid=grid,
        in_specs=[pl.BlockSpec((BLK, F), lambda i: (i, 0)),
                  pl.BlockSpec((F, F), lambda i: (0, 0)),
                  pl.BlockSpec((F, C), lambda i: (0, 0)),
                  pl.BlockSpec((F, C), lambda i: (0, 0))],
        out_specs=(pl.BlockSpec((BLK, F), lambda i: (i, 0)),
                   pl.BlockSpec((BLK, C), lambda i: (i, 0)),
                   pl.BlockSpec((BLK, C), lambda i: (i, 0))),
        out_shape=(jax.ShapeDtypeStruct((NPAD, F), jnp.float32),
                   jax.ShapeDtypeStruct((NPAD, C), jnp.float32),
                   jax.ShapeDtypeStruct((NPAD, C), jnp.float32)),
    )(xp, W, As, Ad)


def _tc_combine_body(p0_ref, p1_ref, inv_ref):
    tot = p0_ref[...] + p1_ref[...]
    inv_ref[...] = 1.0 / (tot + 1e-16)


def _tc_combine(p0, p1):
    grid = (NPAD // BLK,)
    spec = pl.BlockSpec((BLK, C), lambda i: (i, 0))
    return pl.pallas_call(
        _tc_combine_body,
        grid=grid,
        in_specs=[spec, spec],
        out_specs=spec,
        out_shape=jax.ShapeDtypeStruct((NPAD, C), jnp.float32),
    )(p0, p1)


def _gat_epilogue(a0, a1, b, g, be):
    gt = a0 + a1 + b
    gt = jnp.where(gt >= 0.0, gt, 0.01 * gt)
    m = jnp.mean(gt, axis=1, keepdims=True)
    d = gt - m
    v = jnp.mean(d * d, axis=1, keepdims=True)
    return d * lax.rsqrt(v + 1e-5) * g + be


def _tc_finish_project_body(a0_ref, a1_ref, b_ref, g_ref, be_ref, w_ref,
                            as_ref, ad_ref, h_ref, asn_ref, adn_ref):
    gn = _gat_epilogue(a0_ref[...], a1_ref[...], b_ref[...], g_ref[...],
                       be_ref[...])
    h = jnp.dot(gn, w_ref[...], preferred_element_type=jnp.float32)
    h_ref[...] = h
    asn_ref[...] = jnp.dot(h, as_ref[...], preferred_element_type=jnp.float32)
    adn_ref[...] = jnp.dot(h, ad_ref[...], preferred_element_type=jnp.float32)


def _tc_finish_project(a0, a1, b, g, be, W, As, Ad):
    grid = (NPAD // BLK,)
    rowf = pl.BlockSpec((BLK, F), lambda i: (i, 0))
    one = pl.BlockSpec((1, F), lambda i: (0, 0))
    return pl.pallas_call(
        _tc_finish_project_body,
        grid=grid,
        in_specs=[rowf, rowf, one, one, one,
                  pl.BlockSpec((F, F), lambda i: (0, 0)),
                  pl.BlockSpec((F, C), lambda i: (0, 0)),
                  pl.BlockSpec((F, C), lambda i: (0, 0))],
        out_specs=(rowf,
                   pl.BlockSpec((BLK, C), lambda i: (i, 0)),
                   pl.BlockSpec((BLK, C), lambda i: (i, 0))),
        out_shape=(jax.ShapeDtypeStruct((NPAD, F), jnp.float32),
                   jax.ShapeDtypeStruct((NPAD, C), jnp.float32),
                   jax.ShapeDtypeStruct((NPAD, C), jnp.float32)),
    )(a0, a1, b, g, be, W, As, Ad)


def _tc_decoder_body(a0_ref, a1_ref, b_ref, g_ref, be_ref, wd1_ref, bd1_ref,
                     wd2_ref, bd2_ref, xo_ref, rec_ref):
    gn = _gat_epilogue(a0_ref[...], a1_ref[...], b_ref[...], g_ref[...],
                       be_ref[...])
    xo = 1.0 / (1.0 + jnp.exp(-gn))
    xo_ref[...] = xo
    d1 = jnp.dot(xo, wd1_ref[...], preferred_element_type=jnp.float32)
    d1 = jnp.maximum(d1 + bd1_ref[...], 0.0)
    rec_ref[...] = jnp.dot(d1, wd2_ref[...],
                           preferred_element_type=jnp.float32) + bd2_ref[...]


def _tc_decoder(a0, a1, b, g, be, Wd1, bd1, Wd2, bd2):
    grid = (NPAD // BLK,)
    rowf = pl.BlockSpec((BLK, F), lambda i: (i, 0))
    one = pl.BlockSpec((1, F), lambda i: (0, 0))
    return pl.pallas_call(
        _tc_decoder_body,
        grid=grid,
        in_specs=[rowf, rowf, one, one, one,
                  pl.BlockSpec((F, 2 * F), lambda i: (0, 0)),
                  pl.BlockSpec((1, 2 * F), lambda i: (0, 0)),
                  pl.BlockSpec((2 * F, F), lambda i: (0, 0)),
                  pl.BlockSpec((1, F), lambda i: (0, 0))],
        out_specs=(rowf, rowf),
        out_shape=(jax.ShapeDtypeStruct((NPAD, F), jnp.float32),
                   jax.ShapeDtypeStruct((NPAD, F), jnp.float32)),
    )(a0, a1, b, g, be, Wd1, bd1, Wd2, bd2)


def _expand_attn(a):
    A = (jnp.eye(H, dtype=jnp.float32)[:, None, :] * a[:, :, None])
    A = A.reshape(F, H)
    return jnp.pad(A, ((0, 0), (0, C - H)))


def kernel(x, edge_index, W1, a_src1, a_dst1, b1, W2, a_src2, a_dst2, b2,
           g1, beta1, g2, beta2, Wm, bm, Wd1, bd1, Wd2, bd2):
    f32 = jnp.float32
    src = edge_index[0].astype(jnp.int32)
    dst = edge_index[1].astype(jnp.int32)
    E = src.shape[0]
    EA = E + N
    loops = jnp.arange(N, dtype=jnp.int32)
    padv = N + (jnp.arange(EP - EA, dtype=jnp.int32) % (NPAD - N))
    srcall = jnp.concatenate([src, loops, padv])
    dstall = jnp.concatenate([dst, loops, padv])
    src2 = srcall.reshape(EPC, CH)
    dst2 = dstall.reshape(EPC, CH)
    src2b = srcall.reshape(EPC2, CH2)
    dst2b = dstall.reshape(EPC2, CH2)
    xp = jnp.pad(x.astype(f32), ((0, NPAD - N), (0, 0)))

    As1 = _expand_attn(a_src1.astype(f32))
    Ad1 = _expand_attn(a_dst1.astype(f32))
    As2 = _expand_attn(a_src2.astype(f32))
    Ad2 = _expand_attn(a_dst2.astype(f32))
    z16 = jnp.zeros((RPT, C), f32)
    z128 = jnp.zeros((RPT, F), f32)
    b1r = b1.astype(f32).reshape(1, F)
    g1r = g1.astype(f32).reshape(1, F)
    be1r = beta1.astype(f32).reshape(1, F)
    b2r = b2.astype(f32).reshape(1, F)
    g2r = g2.astype(f32).reshape(1, F)
    be2r = beta2.astype(f32).reshape(1, F)
    bd1r = bd1.astype(f32).reshape(1, 2 * F)
    bd2r = bd2.astype(f32).reshape(1, F)

    h1, as1, ad1 = _tc_project(xp, W1.astype(f32), As1, Ad1)
    e1, esp1 = _sc_edge_logits(src2, dst2, as1, ad1, z16)
    inv1 = _tc_combine(esp1[0], esp1[1])
    alpha1e, accp1 = _sc_aggregate(src2b, dst2b, e1, inv1, h1, z128)
    h2, as2, ad2 = _tc_finish_project(accp1[0], accp1[1], b1r, g1r, be1r,
                                      W2.astype(f32), As2, Ad2)
    e2, esp2 = _sc_edge_logits(src2, dst2, as2, ad2, z16)
    inv2 = _tc_combine(esp2[0], esp2[1])
    alpha2e, accp2 = _sc_aggregate(src2b, dst2b, e2, inv2, h2, z128)

    xo, rec = _tc_decoder(accp2[0], accp2[1], b2r, g2r, be2r,
                          Wd1.astype(f32), bd1r, Wd2.astype(f32), bd2r)

    alpha1 = alpha1e[:EA]
    alpha2 = alpha2e[:EA]
    return (xo[:N], rec[:N], alpha1, alpha2)


# speedup vs baseline: 1.0695x; 1.0695x over previous
import functools

import jax
import jax.numpy as jnp
from jax import lax
from jax.experimental import pallas as pl
from jax.experimental.pallas import tpu as pltpu
from jax.experimental.pallas import tpu_sc as plsc

N = 10000
F = 128
H = 8
C = 16
NPAD = 10240
NT = 16
RPT = NPAD // NT
CH = 128
NW = 32
PW = 84
NB = 4
EPC = PW * NW
EP = EPC * CH
BLK = 512

_SC_PARAMS = dict(
    compiler_params=pltpu.CompilerParams(use_tc_tiling_on_sc=False,
                                         needs_layout_passes=False),
)


def _mesh():
    return plsc.VectorSubcoreMesh(core_axis_name="c", subcore_axis_name="s")


def _sc_edge_logits(src2, dst2, asn, adn, z16):
    @functools.partial(
        pl.kernel,
        out_type=(jax.ShapeDtypeStruct((EP, C), jnp.float32),
                  jax.ShapeDtypeStruct((2, NPAD, C), jnp.float32)),
        mesh=_mesh(),
        scratch_types=[
            pltpu.VMEM((PW, CH), jnp.int32),
            pltpu.VMEM((PW, CH), jnp.int32),
            pltpu.VMEM((NB, CH, C), jnp.float32),
            pltpu.VMEM((NB, CH, C), jnp.float32),
            pltpu.VMEM_SHARED((NPAD, C), jnp.float32),
            pltpu.SemaphoreType.DMA((NB,)),
            pltpu.SemaphoreType.DMA((NB,)),
            pltpu.SemaphoreType.DMA((NB,)),
            pltpu.SemaphoreType.DMA((NB,)),
        ],
        **_SC_PARAMS,
    )
    def k(src_h, dst_h, as_h, ad_h, z_h, e_h, esum_h, isrc, idst, ba, bb,
          acc, sga, sgb, sse, ssc):
        cid = lax.axis_index("c")
        sid = lax.axis_index("s")
        wid = sid * 2 + cid
        pltpu.sync_copy(z_h, acc.at[pl.ds(sid * RPT, RPT)])
        pltpu.sync_copy(src_h.at[pl.ds(wid * PW, PW)], isrc)
        pltpu.sync_copy(dst_h.at[pl.ds(wid * PW, PW)], idst)
        plsc.subcore_barrier()

        def issue_g(j, k_):
            pltpu.async_copy(as_h.at[isrc.at[j]], ba.at[k_], sga.at[k_])
            pltpu.async_copy(ad_h.at[idst.at[j]], bb.at[k_], sgb.at[k_])

        def wait_g(j, k_):
            pltpu.make_async_copy(as_h.at[isrc.at[j]], ba.at[k_],
                                  sga.at[k_]).wait()
            pltpu.make_async_copy(ad_h.at[idst.at[j]], bb.at[k_],
                                  sgb.at[k_]).wait()

        def issue_s(j, k_):
            base = pl.multiple_of((wid * PW + j) * CH, CH)
            pltpu.async_copy(ba.at[k_], e_h.at[pl.ds(base, CH)],
                             sse.at[k_])
            pltpu.async_copy(ba.at[k_], acc.at[idst.at[j]], ssc.at[k_],
                             add=True)

        def wait_s(j, k_):
            base = pl.multiple_of((wid * PW + j) * CH, CH)
            pltpu.make_async_copy(ba.at[k_], e_h.at[pl.ds(base, CH)],
                                  sse.at[k_]).wait()
            pltpu.make_async_copy(ba.at[k_], acc.at[idst.at[j]],
                                  ssc.at[k_]).wait()

        def compute(k_):
            def row_body(r, _):
                v = ba[k_, r, :] + bb[k_, r, :]
                v = jnp.where(v >= 0.0, v, v * 0.2)
                ba[k_, r, :] = jnp.exp(v)
                return 0

            lax.fori_loop(0, CH, row_body, 0, unroll=4)

        def step(j, k_, drain_prev, prefetch):
            wait_g(j, k_)
            compute(k_)
            issue_s(j, k_)
            k2 = (k_ + 2) % NB
            if drain_prev:
                wait_s(j - 2, k2)
            if prefetch:
                issue_g(j + 2, k2)

        issue_g(0, 0)
        issue_g(1, 1)
        for k_ in range(NB):
            step(k_, k_, k_ >= 2, True)

        def outer(i, _):
            for k_ in range(NB):
                step(i * NB + k_, k_, True, True)
            return 0

        lax.fori_loop(1, PW // NB - 1, outer, 0)
        for k_ in range(NB):
            j = PW - NB + k_
            step(j, k_, True, j + 2 < PW)
        wait_s(PW - 2, (PW - 2) % NB)
        wait_s(PW - 1, (PW - 1) % NB)
        plsc.subcore_barrier()
        pltpu.sync_copy(acc.at[pl.ds(sid * RPT, RPT)],
                        esum_h.at[cid, pl.ds(sid * RPT, RPT)])

    return k(src2, dst2, asn, adn, z16)


CH2 = 64
PW2 = PW * 2
EPC2 = EPC * 2


def _sc_aggregate(src2, dst2, e_all, invn, hn, z128):
    @functools.partial(
        pl.kernel,
        out_type=(jax.ShapeDtypeStruct((EP, C), jnp.float32),
                  jax.ShapeDtypeStruct((2, NPAD, F), jnp.float32)),
        mesh=_mesh(),
        scratch_types=[
            pltpu.VMEM((4, CH2), jnp.int32),
            pltpu.VMEM((4, CH2), jnp.int32),
            pltpu.VMEM((3, CH2, C), jnp.float32),
            pltpu.VMEM((3, CH2, C), jnp.float32),
            pltpu.VMEM((3, CH2, F), jnp.float32),
            pltpu.VMEM_SHARED((NPAD, F), jnp.float32),
            pltpu.SemaphoreType.DMA((4,)),
            pltpu.SemaphoreType.DMA((4,)),
            pltpu.SemaphoreType.DMA((3,)),
            pltpu.SemaphoreType.DMA((3,)),
            pltpu.SemaphoreType.DMA((3,)),
            pltpu.SemaphoreType.DMA((3,)),
            pltpu.SemaphoreType.DMA((3,)),
        ],
        **_SC_PARAMS,
    )
    def k(src_h, dst_h, e_h, inv_h, h_h, z_h, alpha_h, accp_h, isrc, idst,
          be, binv, brows, acc, sis, sid_, sge, sgi, sgh, ssa, ssc):
        cid = lax.axis_index("c")
        sid = lax.axis_index("s")
        wid = sid * 2 + cid
        pltpu.sync_copy(z_h, acc.at[pl.ds(sid * RPT, RPT)])
        plsc.subcore_barrier()

        def issue_i(j, q):
            g = wid * PW2 + j
            pltpu.async_copy(src_h.at[g], isrc.at[q], sis.at[q])
            pltpu.async_copy(dst_h.at[g], idst.at[q], sid_.at[q])

        def wait_i(j, q):
            g = wid * PW2 + j
            pltpu.make_async_copy(src_h.at[g], isrc.at[q], sis.at[q]).wait()
            pltpu.make_async_copy(dst_h.at[g], idst.at[q],
                                  sid_.at[q]).wait()

        def issue_g(j, s, q):
            base = pl.multiple_of((wid * PW2 + j) * CH2, CH2)
            pltpu.async_copy(e_h.at[pl.ds(base, CH2)], be.at[s], sge.at[s])
            pltpu.async_copy(inv_h.at[idst.at[q]], binv.at[s], sgi.at[s])
            pltpu.async_copy(h_h.at[isrc.at[q]], brows.at[s], sgh.at[s])

        def wait_g(j, s, q):
            base = pl.multiple_of((wid * PW2 + j) * CH2, CH2)
            pltpu.make_async_copy(e_h.at[pl.ds(base, CH2)], be.at[s],
                                  sge.at[s]).wait()
            pltpu.make_async_copy(inv_h.at[idst.at[q]], binv.at[s],
                                  sgi.at[s]).wait()
            pltpu.make_async_copy(h_h.at[isrc.at[q]], brows.at[s],
                                  sgh.at[s]).wait()

        def issue_s(j, s, q):
            base = pl.multiple_of((wid * PW2 + j) * CH2, CH2)
            pltpu.async_copy(be.at[s], alpha_h.at[pl.ds(base, CH2)],
                             ssa.at[s])
            pltpu.async_copy(brows.at[s], acc.at[idst.at[q]], ssc.at[s],
                             add=True)

        def wait_s(j, s, q):
            base = pl.multiple_of((wid * PW2 + j) * CH2, CH2)
            pltpu.make_async_copy(be.at[s], alpha_h.at[pl.ds(base, CH2)],
                                  ssa.at[s]).wait()
            pltpu.make_async_copy(brows.at[s], acc.at[idst.at[q]],
                                  ssc.at[s]).wait()

        gdn = lax.GatherDimensionNumbers(offset_dims=(),
                                         collapsed_slice_dims=(0,),
                                         start_index_map=(0,))

        def compute(s):
            def row_body(r, _):
                va = be[s, r, :] * binv[s, r, :]
                be[s, r, :] = va
                for hh in range(H):
                    ah = lax.gather(
                        va, jnp.full((C, 1), hh, jnp.int32), gdn, (1,),
                        mode=lax.GatherScatterMode.PROMISE_IN_BOUNDS)
                    brows[s, r, pl.ds(hh * C, C)] = (
                        brows[s, r, pl.ds(hh * C, C)] * ah)
                return 0

            lax.fori_loop(0, CH2, row_body, 0, unroll=2)

        def step(j, k_, drain_prev, pf_g, pf_i):
            s = k_ % 3
            s1 = (k_ + 1) % 3
            q = k_ % 4
            q1 = (k_ + 1) % 4
            q2 = (k_ + 2) % 4
            wait_g(j, s, q)
            if pf_g:
                wait_i(j + 1, q1)
            if drain_prev:
                wait_s(j - 2, s1, q2)
            if pf_g:
                issue_g(j + 1, s1, q1)
            compute(s)
            issue_s(j, s, q)
            if pf_i:
                issue_i(j + 2, q2)

        issue_i(0, 0)
        issue_i(1, 1)
        wait_i(0, 0)
        issue_g(0, 0, 0)
        for k_ in range(12):
            step(k_, k_, k_ >= 2, True, True)

        def outer(i, _):
            for k_ in range(12):
                step(i * 12 + k_, k_, True, True, True)
            return 0

        lax.fori_loop(1, PW2 // 12 - 1, outer, 0)
        for k_ in range(12):
            j = PW2 - 12 + k_
            step(j, k_, True, j + 1 < PW2, j + 2 < PW2)
        wait_s(PW2 - 2, (PW2 - 2) % 3, (PW2 - 2) % 4)
        wait_s(PW2 - 1, (PW2 - 1) % 3, (PW2 - 1) % 4)
        plsc.subcore_barrier()
        pltpu.sync_copy(acc.at[pl.ds(sid * RPT, RPT)],
                        accp_h.at[cid, pl.ds(sid * RPT, RPT)])

    return k(src2, dst2, e_all, invn, hn, z128)


def _tc_project_body(x_ref, w_ref, as_ref, ad_ref, h_ref, asn_ref, adn_ref):
    h = jnp.dot(x_ref[...], w_ref[...], preferred_element_type=jnp.float32)
    h_ref[...] = h
    asn_ref[...] = jnp.dot(h, as_ref[...], preferred_element_type=jnp.float32)
    adn_ref[...] = jnp.dot(h, ad_ref[...], preferred_element_type=jnp.float32)


def _tc_project(xp, W, As, Ad):
    grid = (NPAD // BLK,)
    return pl.pallas_call(
        _tc_project_body,
        grid=grid,
        in_specs=[pl.BlockSpec((BLK, F), lambda i: (i, 0)),
                  pl.BlockSpec((F, F), lambda i: (0, 0)),
                  pl.BlockSpec((F, C), lambda i: (0, 0)),
                  pl.BlockSpec((F, C), lambda i: (0, 0))],
        out_specs=(pl.BlockSpec((BLK, F), lambda i: (i, 0)),
                   pl.BlockSpec((BLK, C), lambda i: (i, 0)),
                   pl.BlockSpec((BLK, C), lambda i: (i, 0))),
        out_shape=(jax.ShapeDtypeStruct((NPAD, F), jnp.float32),
                   jax.ShapeDtypeStruct((NPAD, C), jnp.float32),
                   jax.ShapeDtypeStruct((NPAD, C), jnp.float32)),
    )(xp, W, As, Ad)


def _tc_combine_body(p0_ref, p1_ref, inv_ref):
    tot = p0_ref[...] + p1_ref[...]
    inv_ref[...] = 1.0 / (tot + 1e-16)


def _tc_combine(p0, p1):
    grid = (NPAD // BLK,)
    spec = pl.BlockSpec((BLK, C), lambda i: (i, 0))
    return pl.pallas_call(
        _tc_combine_body,
        grid=grid,
        in_specs=[spec, spec],
        out_specs=spec,
        out_shape=jax.ShapeDtypeStruct((NPAD, C), jnp.float32),
    )(p0, p1)


def _gat_epilogue(a0, a1, b, g, be):
    gt = a0 + a1 + b
    gt = jnp.where(gt >= 0.0, gt, 0.01 * gt)
    m = jnp.mean(gt, axis=1, keepdims=True)
    d = gt - m
    v = jnp.mean(d * d, axis=1, keepdims=True)
    return d * lax.rsqrt(v + 1e-5) * g + be


def _tc_finish_project_body(a0_ref, a1_ref, b_ref, g_ref, be_ref, w_ref,
                            as_ref, ad_ref, h_ref, asn_ref, adn_ref):
    gn = _gat_epilogue(a0_ref[...], a1_ref[...], b_ref[...], g_ref[...],
                       be_ref[...])
    h = jnp.dot(gn, w_ref[...], preferred_element_type=jnp.float32)
    h_ref[...] = h
    asn_ref[...] = jnp.dot(h, as_ref[...], preferred_element_type=jnp.float32)
    adn_ref[...] = jnp.dot(h, ad_ref[...], preferred_element_type=jnp.float32)


def _tc_finish_project(a0, a1, b, g, be, W, As, Ad):
    grid = (NPAD // BLK,)
    rowf = pl.BlockSpec((BLK, F), lambda i: (i, 0))
    one = pl.BlockSpec((1, F), lambda i: (0, 0))
    return pl.pallas_call(
        _tc_finish_project_body,
        grid=grid,
        in_specs=[rowf, rowf, one, one, one,
                  pl.BlockSpec((F, F), lambda i: (0, 0)),
                  pl.BlockSpec((F, C), lambda i: (0, 0)),
                  pl.BlockSpec((F, C), lambda i: (0, 0))],
        out_specs=(rowf,
                   pl.BlockSpec((BLK, C), lambda i: (i, 0)),
                   pl.BlockSpec((BLK, C), lambda i: (i, 0))),
        out_shape=(jax.ShapeDtypeStruct((NPAD, F), jnp.float32),
                   jax.ShapeDtypeStruct((NPAD, C), jnp.float32),
                   jax.ShapeDtypeStruct((NPAD, C), jnp.float32)),
    )(a0, a1, b, g, be, W, As, Ad)


def _tc_decoder_body(a0_ref, a1_ref, b_ref, g_ref, be_ref, wd1_ref, bd1_ref,
                     wd2_ref, bd2_ref, xo_ref, rec_ref):
    gn = _gat_epilogue(a0_ref[...], a1_ref[...], b_ref[...], g_ref[...],
                       be_ref[...])
    xo = 1.0 / (1.0 + jnp.exp(-gn))
    xo_ref[...] = xo
    d1 = jnp.dot(xo, wd1_ref[...], preferred_element_type=jnp.float32)
    d1 = jnp.maximum(d1 + bd1_ref[...], 0.0)
    rec_ref[...] = jnp.dot(d1, wd2_ref[...],
                           preferred_element_type=jnp.float32) + bd2_ref[...]


def _tc_decoder(a0, a1, b, g, be, Wd1, bd1, Wd2, bd2):
    grid = (NPAD // BLK,)
    rowf = pl.BlockSpec((BLK, F), lambda i: (i, 0))
    one = pl.BlockSpec((1, F), lambda i: (0, 0))
    return pl.pallas_call(
        _tc_decoder_body,
        grid=grid,
        in_specs=[rowf, rowf, one, one, one,
                  pl.BlockSpec((F, 2 * F), lambda i: (0, 0)),
                  pl.BlockSpec((1, 2 * F), lambda i: (0, 0)),
                  pl.BlockSpec((2 * F, F), lambda i: (0, 0)),
                  pl.BlockSpec((1, F), lambda i: (0, 0))],
        out_specs=(rowf, rowf),
        out_shape=(jax.ShapeDtypeStruct((NPAD, F), jnp.float32),
                   jax.ShapeDtypeStruct((NPAD, F), jnp.float32)),
    )(a0, a1, b, g, be, Wd1, bd1, Wd2, bd2)


def _expand_attn(a):
    A = (jnp.eye(H, dtype=jnp.float32)[:, None, :] * a[:, :, None])
    A = A.reshape(F, H)
    return jnp.pad(A, ((0, 0), (0, C - H)))


def kernel(x, edge_index, W1, a_src1, a_dst1, b1, W2, a_src2, a_dst2, b2,
           g1, beta1, g2, beta2, Wm, bm, Wd1, bd1, Wd2, bd2):
    f32 = jnp.float32
    src = edge_index[0].astype(jnp.int32)
    dst = edge_index[1].astype(jnp.int32)
    E = src.shape[0]
    EA = E + N
    loops = jnp.arange(N, dtype=jnp.int32)
    padv = N + (jnp.arange(EP - EA, dtype=jnp.int32) % (NPAD - N))
    srcall = jnp.concatenate([src, loops, padv])
    dstall = jnp.concatenate([dst, loops, padv])
    src2 = srcall.reshape(EPC, CH)
    dst2 = dstall.reshape(EPC, CH)
    src2b = srcall.reshape(EPC2, CH2)
    dst2b = dstall.reshape(EPC2, CH2)
    xp = jnp.pad(x.astype(f32), ((0, NPAD - N), (0, 0)))

    As1 = _expand_attn(a_src1.astype(f32))
    Ad1 = _expand_attn(a_dst1.astype(f32))
    As2 = _expand_attn(a_src2.astype(f32))
    Ad2 = _expand_attn(a_dst2.astype(f32))
    z16 = jnp.zeros((RPT, C), f32)
    z128 = jnp.zeros((RPT, F), f32)
    b1r = b1.astype(f32).reshape(1, F)
    g1r = g1.astype(f32).reshape(1, F)
    be1r = beta1.astype(f32).reshape(1, F)
    b2r = b2.astype(f32).reshape(1, F)
    g2r = g2.astype(f32).reshape(1, F)
    be2r = beta2.astype(f32).reshape(1, F)
    bd1r = bd1.astype(f32).reshape(1, 2 * F)
    bd2r = bd2.astype(f32).reshape(1, F)

    h1, as1, ad1 = _tc_project(xp, W1.astype(f32), As1, Ad1)
    e1, esp1 = _sc_edge_logits(src2, dst2, as1, ad1, z16)
    inv1 = _tc_combine(esp1[0], esp1[1])
    alpha1e, accp1 = _sc_aggregate(src2b, dst2b, e1, inv1, h1, z128)
    h2, as2, ad2 = _tc_finish_project(accp1[0], accp1[1], b1r, g1r, be1r,
                                      W2.astype(f32), As2, Ad2)
    e2, esp2 = _sc_edge_logits(src2, dst2, as2, ad2, z16)
    inv2 = _tc_combine(esp2[0], esp2[1])
    alpha2e, accp2 = _sc_aggregate(src2b, dst2b, e2, inv2, h2, z128)

    xo, rec = _tc_decoder(accp2[0], accp2[1], b2r, g2r, be2r,
                          Wd1.astype(f32), bd1r, Wd2.astype(f32), bd2r)

    alpha1 = alpha1e[:EA, :H]
    alpha2 = alpha2e[:EA, :H]
    return (xo[:N], rec[:N], alpha1, alpha2)
